# baseline (speedup 1.0000x reference)
import functools

import jax
import jax.numpy as jnp
from jax import lax
from jax.experimental import pallas as pl
from jax.experimental.pallas import tpu as pltpu
from jax.experimental.pallas import tpu_sc as plsc

VOCAB = 1000
B, T = 1024, 50
NTOK = B * T
NC, NS = 2, 16
NW = NC * NS
C8 = VOCAB // 8
NCU = 8
NBT = B // 128
NTASK = T * NCU * NBT
TASKS_PER = NTASK // NW

_mesh = plsc.VectorSubcoreMesh(core_axis_name="c", subcore_axis_name="s")


def _lse_body(t_ref, out_ref):
    t = t_ref[...]
    m = jnp.max(t, axis=1)
    s = jnp.sum(jnp.exp(t - m[:, None]), axis=1)
    out_ref[0, 0, :] = m + jnp.log(s)


def _row_lse(table):
    out = pl.pallas_call(
        _lse_body,
        grid=(5,),
        in_specs=[pl.BlockSpec((200, VOCAB), lambda i: (i, 0))],
        out_specs=pl.BlockSpec((1, 1, 200), lambda i: (i, 0, 0)),
        out_shape=jax.ShapeDtypeStruct((5, 1, 200), jnp.float32),
    )(table)
    return jnp.pad(out.reshape(VOCAB), (0, 1024 - VOCAB))


@functools.partial(
    pl.kernel,
    mesh=_mesh,
    compiler_params=pltpu.CompilerParams(
        needs_layout_passes=False, use_tc_tiling_on_sc=False),
    out_type=[
        jax.ShapeDtypeStruct((T, C8, NBT, 8, 128), jnp.float32),
        jax.ShapeDtypeStruct((NW, 16), jnp.float32),
    ],
    scratch_types=[
        pltpu.VMEM((128,), jnp.int32),
        pltpu.VMEM((128,), jnp.int32),
        pltpu.VMEM((128,), jnp.int32),
        pltpu.VMEM((128,), jnp.int32),
        pltpu.VMEM((128,), jnp.int32),
        pltpu.VMEM((128,), jnp.int32),
        pltpu.VMEM((1024,), jnp.float32),
        pltpu.VMEM((128, 128), jnp.float32),
        pltpu.VMEM((128, 128), jnp.float32),
        pltpu.VMEM((16, 8, 128), jnp.float32),
        pltpu.VMEM((16, 8, 128), jnp.float32),
        pltpu.VMEM((16,), jnp.float32),
        pltpu.SemaphoreType.DMA,
        pltpu.SemaphoreType.DMA,
        pltpu.SemaphoreType.DMA,
        pltpu.SemaphoreType.DMA,
        pltpu.SemaphoreType.DMA,
        pltpu.SemaphoreType.DMA,
        pltpu.SemaphoreType.DMA,
        pltpu.SemaphoreType.DMA,
    ],
)
def _gather_loss(tab8_hbm, idxt_hbm, tgtt_hbm, lse_hbm, out_hbm, part_hbm,
                 idxv0, idxv1, tgtv0, tgtv1, pidx0, pidx1, lse_v,
                 g0, g1, t0, t1, acc_v,
                 gsem0, gsem1, wsem0, wsem1, ism0, ism1, tsm0, tsm1):
    wid = lax.axis_index("s") * NC + lax.axis_index("c")
    pltpu.sync_copy(lse_hbm, lse_v)
    lanes = lax.iota(jnp.int32, 16)
    idxvs, tgtvs = (idxv0, idxv1), (tgtv0, tgtv1)
    pidxs, gs, ts = (pidx0, pidx1), (g0, g1), (t0, t1)
    gsems, wsems = (gsem0, gsem1), (wsem0, wsem1)
    isms, tsms = (ism0, ism1), (tsm0, tsm1)
    rows = [lanes + j * 16 for j in range(8)]

    def tinfo(kk):
        tid = wid * TASKS_PER + kk
        return tid // (NCU * NBT), (tid // NCU) % NBT, tid % NCU

    def idx_copies(kk, s):
        tpos, bt, _ = tinfo(kk)
        src_i = idxt_hbm.at[tpos, pl.ds(bt * 128, 128)]
        src_t = tgtt_hbm.at[tpos, pl.ds(bt * 128, 128)]
        return (pltpu.make_async_copy(src_i, idxvs[s], isms[s]),
                pltpu.make_async_copy(src_t, tgtvs[s], tsms[s]))

    def start_idx(kk, s):
        for c in idx_copies(kk, s):
            c.start()

    def wait_idx(kk, s):
        for c in idx_copies(kk, s):
            c.wait()

    def compute_pidx(s, cu):
        for j in range(8):
            pidxs[s][pl.ds(j * 16, 16)] = idxvs[s][pl.ds(j * 16, 16)] * 8 + cu

    def gather_copy(s):
        return pltpu.make_async_copy(tab8_hbm.at[pidxs[s]], gs[s], gsems[s])

    def write_copies(kk, s):
        tpos, bt, cu = tinfo(kk)
        full = pltpu.make_async_copy(
            ts[s], out_hbm.at[tpos, pl.ds(cu * 16, 16), bt], wsems[s])
        part = pltpu.make_async_copy(
            ts[s].at[pl.ds(0, 13)],
            out_hbm.at[tpos, pl.ds(cu * 16, 13), bt], wsems[s])
        return cu, full, part

    def start_write(kk, s):
        cu, full, part = write_copies(kk, s)
        pl.when(cu < NCU - 1)(lambda: full.start())
        pl.when(cu == NCU - 1)(lambda: part.start())

    def wait_write(kk, s):
        cu, full, part = write_copies(kk, s)
        pl.when(cu < NCU - 1)(lambda: full.wait())
        pl.when(cu == NCU - 1)(lambda: part.wait())

    def loss(s, cu, acc):
        for j in range(8):
            iv = idxvs[s][pl.ds(j * 16, 16)]
            tv = tgtvs[s][pl.ds(j * 16, 16)]
            e16 = plsc.load_gather(gs[s], [rows[j], tv & 127])
            l16 = plsc.load_gather(lse_v, [iv])
            acc = acc + jnp.where((tv >> 7) == cu, l16 - e16, 0.0)
        return acc

    def transpose(s):
        def tr(cc0, _):
            for j in range(8):
                lv = rows[j]
                m = (cc0 + lv) & 127
                v = plsc.load_gather(gs[s], [lv, m])
                plsc.store_scatter(ts[s], [m >> 3, m & 7, lv], v)
            return 0
        lax.fori_loop(0, 128, tr, 0)

    def one_task(k, s, acc):
        _, _, cu = tinfo(k)
        gather_copy(s).wait()
        acc = loss(s, cu, acc)

        @pl.when(k < TASKS_PER - 1)
        def _():
            wait_idx(k + 1, s ^ 1)
            compute_pidx(s ^ 1, (cu + 1) % NCU)
            gather_copy(s ^ 1).start()

        @pl.when(k < TASKS_PER - 2)
        def _():
            start_idx(k + 2, s)

        @pl.when(k >= 2)
        def _():
            wait_write(k - 2, s)

        transpose(s)
        start_write(k, s)
        return acc

    start_idx(0, 0)
    wait_idx(0, 0)
    _, _, cu0 = tinfo(0)
    compute_pidx(0, cu0)
    gather_copy(0).start()
    start_idx(1, 1)

    def pair(p, acc):
        acc = one_task(2 * p, 0, acc)
        return one_task(2 * p + 1, 1, acc)

    acc = lax.fori_loop(0, TASKS_PER // 2, pair, jnp.zeros((16,), jnp.float32))
    wait_write(TASKS_PER - 2, 0)
    wait_write(TASKS_PER - 1, 1)
    acc_v[...] = acc
    pltpu.sync_copy(acc_v, part_hbm.at[wid])


def _combine_body(p_ref, o_ref):
    o_ref[...] = (jnp.sum(p_ref[...]) / NTOK).reshape(1, 1)


def _combine(parts):
    out = pl.pallas_call(
        _combine_body,
        out_shape=jax.ShapeDtypeStruct((1, 1), jnp.float32),
    )(parts)
    return out[0, 0]


def kernel(idx, targets, table):
    idxt = idx.astype(jnp.int32).T.reshape(T, B)
    tgtt = targets.astype(jnp.int32).T.reshape(T, B)
    tab8 = jnp.pad(table, ((0, 0), (0, 1024 - VOCAB))).reshape(VOCAB * 8, 128)
    lse = _row_lse(table)
    out7, parts = _gather_loss(tab8, idxt, tgtt, lse)
    logits = out7.transpose(2, 4, 0, 1, 3).reshape(B, T, VOCAB)
    return logits, _combine(parts)


# speedup vs baseline: 2.0609x; 2.0609x over previous
import functools

import jax
import jax.numpy as jnp
from jax import lax
from jax.experimental import pallas as pl
from jax.experimental.pallas import tpu as pltpu
from jax.experimental.pallas import tpu_sc as plsc

VOCAB = 1000
B, T = 1024, 50
NTOK = B * T
NC, NS = 2, 16
NW = NC * NS
C8 = VOCAB // 8
NCU = 8
NBT = B // 128
NTASK = T * NCU * NBT
TASKS_PER = NTASK // NW

_mesh = plsc.VectorSubcoreMesh(core_axis_name="c", subcore_axis_name="s")


def _lse_body(t_ref, out_ref):
    t = t_ref[...]
    m = jnp.max(t, axis=1)
    s = jnp.sum(jnp.exp(t - m[:, None]), axis=1)
    out_ref[0, 0, :] = m + jnp.log(s)


def _row_lse(table):
    out = pl.pallas_call(
        _lse_body,
        grid=(5,),
        in_specs=[pl.BlockSpec((200, VOCAB), lambda i: (i, 0))],
        out_specs=pl.BlockSpec((1, 1, 200), lambda i: (i, 0, 0)),
        out_shape=jax.ShapeDtypeStruct((5, 1, 200), jnp.float32),
    )(table)
    return jnp.pad(out.reshape(VOCAB), (0, 1024 - VOCAB))


@functools.partial(
    pl.kernel,
    mesh=_mesh,
    compiler_params=pltpu.CompilerParams(
        needs_layout_passes=False, use_tc_tiling_on_sc=False),
    out_type=[
        jax.ShapeDtypeStruct((T, C8, NBT, 8, 128), jnp.float32),
        jax.ShapeDtypeStruct((NW, 16), jnp.float32),
    ],
    scratch_types=[
        pltpu.VMEM((128,), jnp.int32),
        pltpu.VMEM((128,), jnp.int32),
        pltpu.VMEM((128,), jnp.int32),
        pltpu.VMEM((128,), jnp.int32),
        pltpu.VMEM((128,), jnp.int32),
        pltpu.VMEM((128,), jnp.int32),
        pltpu.VMEM((1024,), jnp.float32),
        pltpu.VMEM((128, 128), jnp.float32),
        pltpu.VMEM((128, 128), jnp.float32),
        pltpu.VMEM((16, 8, 128), jnp.float32),
        pltpu.VMEM((16, 8, 128), jnp.float32),
        pltpu.VMEM((16,), jnp.float32),
        pltpu.SemaphoreType.DMA,
        pltpu.SemaphoreType.DMA,
        pltpu.SemaphoreType.DMA,
        pltpu.SemaphoreType.DMA,
        pltpu.SemaphoreType.DMA,
        pltpu.SemaphoreType.DMA,
        pltpu.SemaphoreType.DMA,
        pltpu.SemaphoreType.DMA,
    ],
)
def _gather_loss(tab8_hbm, idxt_hbm, tgtt_hbm, lse_hbm, out_hbm, part_hbm,
                 idxv0, idxv1, tgtv0, tgtv1, pidx0, pidx1, lse_v,
                 g0, g1, t0, t1, acc_v,
                 gsem0, gsem1, wsem0, wsem1, ism0, ism1, tsm0, tsm1):
    wid = lax.axis_index("s") * NC + lax.axis_index("c")
    pltpu.sync_copy(lse_hbm, lse_v)
    lanes = lax.iota(jnp.int32, 16)
    idxvs, tgtvs = (idxv0, idxv1), (tgtv0, tgtv1)
    pidxs, gs, ts = (pidx0, pidx1), (g0, g1), (t0, t1)
    gsems, wsems = (gsem0, gsem1), (wsem0, wsem1)
    isms, tsms = (ism0, ism1), (tsm0, tsm1)
    rows = [lanes + j * 16 for j in range(8)]

    def tinfo(kk):
        tid = wid * TASKS_PER + kk
        return tid // (NCU * NBT), (tid // NCU) % NBT, tid % NCU

    def idx_copies(kk, s):
        tpos, bt, _ = tinfo(kk)
        src_i = idxt_hbm.at[tpos, pl.ds(bt * 128, 128)]
        src_t = tgtt_hbm.at[tpos, pl.ds(bt * 128, 128)]
        return (pltpu.make_async_copy(src_i, idxvs[s], isms[s]),
                pltpu.make_async_copy(src_t, tgtvs[s], tsms[s]))

    def start_idx(kk, s):
        for c in idx_copies(kk, s):
            c.start()

    def wait_idx(kk, s):
        for c in idx_copies(kk, s):
            c.wait()

    def compute_pidx(s, cu):
        for j in range(8):
            pidxs[s][pl.ds(j * 16, 16)] = idxvs[s][pl.ds(j * 16, 16)] * 8 + cu

    def gather_copy(s):
        return pltpu.make_async_copy(tab8_hbm.at[pidxs[s]], gs[s], gsems[s])

    def write_copies(kk, s):
        tpos, bt, cu = tinfo(kk)
        full = pltpu.make_async_copy(
            ts[s], out_hbm.at[tpos, pl.ds(cu * 16, 16), bt], wsems[s])
        part = pltpu.make_async_copy(
            ts[s].at[pl.ds(0, 13)],
            out_hbm.at[tpos, pl.ds(cu * 16, 13), bt], wsems[s])
        return cu, full, part

    def start_write(kk, s):
        cu, full, part = write_copies(kk, s)
        pl.when(cu < NCU - 1)(lambda: full.start())
        pl.when(cu == NCU - 1)(lambda: part.start())

    def wait_write(kk, s):
        cu, full, part = write_copies(kk, s)
        pl.when(cu < NCU - 1)(lambda: full.wait())
        pl.when(cu == NCU - 1)(lambda: part.wait())

    def loss(s, cu, acc):
        for j in range(8):
            iv = idxvs[s][pl.ds(j * 16, 16)]
            tv = tgtvs[s][pl.ds(j * 16, 16)]
            e16 = plsc.load_gather(gs[s], [rows[j], tv & 127])
            l16 = plsc.load_gather(lse_v, [iv])
            acc = acc + jnp.where((tv >> 7) == cu, l16 - e16, 0.0)
        return acc

    def transpose(s):
        @plsc.parallel_loop(0, 128, unroll=4)
        def tr(cc0):
            for j in range(8):
                lv = rows[j]
                m = (cc0 + lv) & 127
                v = plsc.load_gather(gs[s], [lv, m])
                plsc.store_scatter(ts[s], [m >> 3, m & 7, lv], v)

    def one_task(k, s, acc):
        _, _, cu = tinfo(k)
        gather_copy(s).wait()
        acc = loss(s, cu, acc)

        @pl.when(k < TASKS_PER - 1)
        def _():
            wait_idx(k + 1, s ^ 1)
            compute_pidx(s ^ 1, (cu + 1) % NCU)
            gather_copy(s ^ 1).start()

        @pl.when(k < TASKS_PER - 2)
        def _():
            start_idx(k + 2, s)

        @pl.when(k >= 2)
        def _():
            wait_write(k - 2, s)

        transpose(s)
        start_write(k, s)
        return acc

    start_idx(0, 0)
    wait_idx(0, 0)
    _, _, cu0 = tinfo(0)
    compute_pidx(0, cu0)
    gather_copy(0).start()
    start_idx(1, 1)

    def pair(p, acc):
        acc = one_task(2 * p, 0, acc)
        return one_task(2 * p + 1, 1, acc)

    acc = lax.fori_loop(0, TASKS_PER // 2, pair, jnp.zeros((16,), jnp.float32))
    wait_write(TASKS_PER - 2, 0)
    wait_write(TASKS_PER - 1, 1)
    acc_v[...] = acc
    pltpu.sync_copy(acc_v, part_hbm.at[wid])


def _combine_body(p_ref, o_ref):
    o_ref[...] = (jnp.sum(p_ref[...]) / NTOK).reshape(1, 1)


def _combine(parts):
    out = pl.pallas_call(
        _combine_body,
        out_shape=jax.ShapeDtypeStruct((1, 1), jnp.float32),
    )(parts)
    return out[0, 0]


def kernel(idx, targets, table):
    idxt = idx.astype(jnp.int32).T.reshape(T, B)
    tgtt = targets.astype(jnp.int32).T.reshape(T, B)
    tab8 = jnp.pad(table, ((0, 0), (0, 1024 - VOCAB))).reshape(VOCAB * 8, 128)
    lse = _row_lse(table)
    out7, parts = _gather_loss(tab8, idxt, tgtt, lse)
    logits = out7.transpose(2, 4, 0, 1, 3).reshape(B, T, VOCAB)
    return logits, _combine(parts)


# speedup vs baseline: 2.5235x; 1.2244x over previous
import functools

import jax
import jax.numpy as jnp
from jax import lax
from jax.experimental import pallas as pl
from jax.experimental.pallas import tpu as pltpu
from jax.experimental.pallas import tpu_sc as plsc

VOCAB = 1000
B, T = 1024, 50
NTOK = B * T
NC, NS = 2, 16
NW = NC * NS
C8 = VOCAB // 8
NCU = 8
NBT = B // 128
NTASK = T * NCU * NBT
TASKS_PER = NTASK // NW

_mesh = plsc.VectorSubcoreMesh(core_axis_name="c", subcore_axis_name="s")


def _lse_body(t_ref, out_ref):
    t = t_ref[...]
    m = jnp.max(t, axis=1)
    s = jnp.sum(jnp.exp(t - m[:, None]), axis=1)
    out_ref[0, 0, :] = m + jnp.log(s)


def _row_lse(table):
    out = pl.pallas_call(
        _lse_body,
        grid=(5,),
        in_specs=[pl.BlockSpec((200, VOCAB), lambda i: (i, 0))],
        out_specs=pl.BlockSpec((1, 1, 200), lambda i: (i, 0, 0)),
        out_shape=jax.ShapeDtypeStruct((5, 1, 200), jnp.float32),
    )(table)
    return jnp.pad(out.reshape(VOCAB), (0, 1024 - VOCAB))


@functools.partial(
    pl.kernel,
    mesh=_mesh,
    compiler_params=pltpu.CompilerParams(
        needs_layout_passes=False, use_tc_tiling_on_sc=False),
    out_type=[
        jax.ShapeDtypeStruct((T, C8, NBT, 8, 128), jnp.float32),
        jax.ShapeDtypeStruct((NW, 16), jnp.float32),
    ],
    scratch_types=[
        pltpu.VMEM((128,), jnp.int32),
        pltpu.VMEM((128,), jnp.int32),
        pltpu.VMEM((128,), jnp.int32),
        pltpu.VMEM((128,), jnp.int32),
        pltpu.VMEM((128,), jnp.int32),
        pltpu.VMEM((128,), jnp.int32),
        pltpu.VMEM((1024,), jnp.float32),
        pltpu.VMEM((128, 128), jnp.float32),
        pltpu.VMEM((128, 128), jnp.float32),
        pltpu.VMEM((16, 8, 128), jnp.float32),
        pltpu.VMEM((16, 8, 128), jnp.float32),
        pltpu.VMEM((16,), jnp.float32),
        pltpu.VMEM_SHARED((VOCAB * 7, 128), jnp.float32),
        pltpu.SemaphoreType.DMA,
        pltpu.SemaphoreType.DMA,
        pltpu.SemaphoreType.DMA,
        pltpu.SemaphoreType.DMA,
        pltpu.SemaphoreType.DMA,
        pltpu.SemaphoreType.DMA,
        pltpu.SemaphoreType.DMA,
        pltpu.SemaphoreType.DMA,
    ],
)
def _gather_loss(tab8_hbm, tab7_hbm, idxt_hbm, tgtt_hbm, lse_hbm,
                 out_hbm, part_hbm,
                 idxv0, idxv1, tgtv0, tgtv1, pidx0, pidx1, lse_v,
                 g0, g1, t0, t1, acc_v, tab_sp,
                 gsem0, gsem1, wsem0, wsem1, ism0, ism1, tsm0, tsm1):
    wid = lax.axis_index("s") * NC + lax.axis_index("c")
    sid = lax.axis_index("s")
    @pl.when(sid < 14)
    def _():
        pltpu.sync_copy(tab7_hbm.at[pl.ds(sid * 500, 500)],
                        tab_sp.at[pl.ds(sid * 500, 500)])
    pltpu.sync_copy(lse_hbm, lse_v)
    plsc.subcore_barrier()
    lanes = lax.iota(jnp.int32, 16)
    idxvs, tgtvs = (idxv0, idxv1), (tgtv0, tgtv1)
    pidxs, gs, ts = (pidx0, pidx1), (g0, g1), (t0, t1)
    gsems, wsems = (gsem0, gsem1), (wsem0, wsem1)
    isms, tsms = (ism0, ism1), (tsm0, tsm1)
    rows = [lanes + j * 16 for j in range(8)]

    def tinfo(kk):
        tid = wid * TASKS_PER + kk
        return tid // (NCU * NBT), (tid // NCU) % NBT, tid % NCU

    def idx_copies(kk, s):
        tpos, bt, _ = tinfo(kk)
        src_i = idxt_hbm.at[tpos, pl.ds(bt * 128, 128)]
        src_t = tgtt_hbm.at[tpos, pl.ds(bt * 128, 128)]
        return (pltpu.make_async_copy(src_i, idxvs[s], isms[s]),
                pltpu.make_async_copy(src_t, tgtvs[s], tsms[s]))

    def start_idx(kk, s):
        for c in idx_copies(kk, s):
            c.start()

    def wait_idx(kk, s):
        for c in idx_copies(kk, s):
            c.wait()

    def compute_pidx(s, cu):
        for j in range(8):
            iv = idxvs[s][pl.ds(j * 16, 16)]
            pidxs[s][pl.ds(j * 16, 16)] = jnp.where(
                cu < NCU - 1, iv * 7 + cu, iv * 8 + 7)

    def gather_sp(s):
        return pltpu.make_async_copy(tab_sp.at[pidxs[s]], gs[s], gsems[s])

    def gather_hbm(s):
        return pltpu.make_async_copy(tab8_hbm.at[pidxs[s]], gs[s], gsems[s])

    def start_gather(s, cu):
        pl.when(cu < NCU - 1)(lambda: gather_sp(s).start())
        pl.when(cu == NCU - 1)(lambda: gather_hbm(s).start())

    def wait_gather(s, cu):
        pl.when(cu < NCU - 1)(lambda: gather_sp(s).wait())
        pl.when(cu == NCU - 1)(lambda: gather_hbm(s).wait())

    def write_copies(kk, s):
        tpos, bt, cu = tinfo(kk)
        full = pltpu.make_async_copy(
            ts[s], out_hbm.at[tpos, pl.ds(cu * 16, 16), bt], wsems[s])
        part = pltpu.make_async_copy(
            ts[s].at[pl.ds(0, 13)],
            out_hbm.at[tpos, pl.ds(cu * 16, 13), bt], wsems[s])
        return cu, full, part

    def start_write(kk, s):
        cu, full, part = write_copies(kk, s)
        pl.when(cu < NCU - 1)(lambda: full.start())
        pl.when(cu == NCU - 1)(lambda: part.start())

    def wait_write(kk, s):
        cu, full, part = write_copies(kk, s)
        pl.when(cu < NCU - 1)(lambda: full.wait())
        pl.when(cu == NCU - 1)(lambda: part.wait())

    def loss(s, cu, acc):
        for j in range(8):
            iv = idxvs[s][pl.ds(j * 16, 16)]
            tv = tgtvs[s][pl.ds(j * 16, 16)]
            e16 = plsc.load_gather(gs[s], [rows[j], tv & 127])
            l16 = plsc.load_gather(lse_v, [iv])
            acc = acc + jnp.where((tv >> 7) == cu, l16 - e16, 0.0)
        return acc

    def transpose(s):
        @plsc.parallel_loop(0, 128, unroll=4)
        def tr(cc0):
            for j in range(8):
                lv = rows[j]
                m = (cc0 + lv) & 127
                v = plsc.load_gather(gs[s], [lv, m])
                plsc.store_scatter(ts[s], [m >> 3, m & 7, lv], v)

    def one_task(k, s, acc):
        _, _, cu = tinfo(k)
        wait_gather(s, cu)
        acc = loss(s, cu, acc)

        @pl.when(k < TASKS_PER - 1)
        def _():
            wait_idx(k + 1, s ^ 1)
            compute_pidx(s ^ 1, (cu + 1) % NCU)
            start_gather(s ^ 1, (cu + 1) % NCU)

        @pl.when(k < TASKS_PER - 2)
        def _():
            start_idx(k + 2, s)

        @pl.when(k >= 2)
        def _():
            wait_write(k - 2, s)

        transpose(s)
        start_write(k, s)
        return acc

    start_idx(0, 0)
    wait_idx(0, 0)
    _, _, cu0 = tinfo(0)
    compute_pidx(0, cu0)
    start_gather(0, cu0)
    start_idx(1, 1)

    def pair(p, acc):
        acc = one_task(2 * p, 0, acc)
        return one_task(2 * p + 1, 1, acc)

    acc = lax.fori_loop(0, TASKS_PER // 2, pair, jnp.zeros((16,), jnp.float32))
    wait_write(TASKS_PER - 2, 0)
    wait_write(TASKS_PER - 1, 1)
    acc_v[...] = acc
    pltpu.sync_copy(acc_v, part_hbm.at[wid])


def _combine_body(p_ref, o_ref):
    o_ref[...] = (jnp.sum(p_ref[...]) / NTOK).reshape(1, 1)


def _combine(parts):
    out = pl.pallas_call(
        _combine_body,
        out_shape=jax.ShapeDtypeStruct((1, 1), jnp.float32),
    )(parts)
    return out[0, 0]


def kernel(idx, targets, table):
    idxt = idx.astype(jnp.int32).T.reshape(T, B)
    tgtt = targets.astype(jnp.int32).T.reshape(T, B)
    tab8 = jnp.pad(table, ((0, 0), (0, 1024 - VOCAB))).reshape(VOCAB * 8, 128)
    tab7 = table[:, :896].reshape(VOCAB * 7, 128)
    lse = _row_lse(table)
    out7, parts = _gather_loss(tab8, tab7, idxt, tgtt, lse)
    logits = out7.transpose(2, 4, 0, 1, 3).reshape(B, T, VOCAB)
    return logits, _combine(parts)
